# Initial kernel scaffold; baseline (speedup 1.0000x reference)
import jax
import jax.numpy as jnp
from jax.experimental import pallas as pl


def kernel(seq_length, key_length, relative_attention_bias):
    raise NotImplementedError("write your pallas kernel here")


# speedup vs baseline: 68.7402x; 68.7402x over previous
import math

import jax
import jax.numpy as jnp
from jax.experimental import pallas as pl
from jax.experimental.pallas import tpu as pltpu

NUM_BUCKETS = 32
NUM_HEADS = 16
MAX_DISTANCE = 128
S_Q = 2048
S_K = 2048
BQ = 128
DIAG_LEN = 4224


def _bucket_values(d, table_ref, h):
    nb = NUM_BUCKETS // 2
    base = jnp.where(d > 0, nb, 0).astype(jnp.int32)
    rp = jnp.abs(d)
    max_exact = nb // 2
    is_small = rp < max_exact
    rpf = rp.astype(jnp.float32)
    large = max_exact + (
        jnp.log(rpf / max_exact)
        / math.log(MAX_DISTANCE / max_exact)
        * (nb - max_exact)
    ).astype(jnp.int32)
    large = jnp.minimum(large, nb - 1)
    bucket = base + jnp.where(is_small, rp, large)
    val = jnp.zeros(d.shape, jnp.float32)
    for b in range(NUM_BUCKETS):
        val = jnp.where(bucket == b, table_ref[b, h], val)
    return val


def _expand_kernel(off_ref, table_ref, out_ref, diag_ref):
    h = pl.program_id(0)
    qb = pl.program_id(1)

    @pl.when(qb == 0)
    def _build():
        r = jax.lax.broadcasted_iota(jnp.int32, (BQ, DIAG_LEN), 0)
        j = jax.lax.broadcasted_iota(jnp.int32, (BQ, DIAG_LEN), 1)
        d = j - 1920 - r + off_ref[0]
        diag_ref[...] = _bucket_values(d, table_ref, h)

    start = pl.multiple_of((15 - qb) * BQ, 128)
    out_ref[0, :, :] = diag_ref[:, pl.ds(start, S_K)]


def kernel(seq_length, key_length, relative_attention_bias):
    off = (jnp.asarray(key_length, jnp.int32) - S_K) - (
        jnp.asarray(seq_length, jnp.int32) - S_Q
    )
    off = off.reshape((1,))
    grid = (NUM_HEADS, S_Q // BQ)
    return pl.pallas_call(
        _expand_kernel,
        grid=grid,
        in_specs=[
            pl.BlockSpec(memory_space=pltpu.SMEM),
            pl.BlockSpec(memory_space=pltpu.SMEM),
        ],
        out_specs=pl.BlockSpec((1, BQ, S_K), lambda h, qb: (h, qb, 0)),
        out_shape=jax.ShapeDtypeStruct((NUM_HEADS, S_Q, S_K), jnp.float32),
        scratch_shapes=[pltpu.VMEM((BQ, DIAG_LEN), jnp.float32)],
    )(off, relative_attention_bias)
